# baseline (speedup 1.0000x reference)
import functools

import jax
import jax.numpy as jnp
from jax import lax
from jax.experimental import pallas as pl
from jax.experimental.pallas import tpu as pltpu
from jax.experimental.pallas import tpu_sc as plsc

N_NODES = 100000
N_EDGES = 6400000
NC, NS = 2, 16
NW = NC * NS
N_PAD = 102400
EPW = N_EDGES // NW
CH = 2000
N_CHUNKS = EPW // CH
CH_H = 4000
N_CHUNKS_H = EPW // CH_H
TILE_N = N_PAD // NS

_mesh = plsc.VectorSubcoreMesh(
    core_axis_name="c", subcore_axis_name="s", num_cores=NC, num_subcores=NS
)


@functools.partial(
    pl.kernel,
    out_type=jax.ShapeDtypeStruct((NW * N_PAD,), jnp.float32),
    mesh=_mesh,
    scratch_types=[
        pltpu.VMEM((N_PAD,), jnp.float32),
        pltpu.VMEM((CH_H,), jnp.int32),
        pltpu.VMEM((CH_H,), jnp.int32),
        pltpu.SemaphoreType.DMA,
        pltpu.SemaphoreType.DMA,
        pltpu.SemaphoreType.DMA,
    ],
    compiler_params=pltpu.CompilerParams(needs_layout_passes=False),
)
def _hist(dst_hbm, zeros_hbm, out_hbm, cnt_v, idx0, idx1, sz, si0, si1):
    c = lax.axis_index("c")
    s = lax.axis_index("s")
    w = c * NS + s
    base = w * EPW
    ibufs = (idx0, idx1)
    isems = (si0, si1)

    def issue_idx(k, p):
        pltpu.async_copy(dst_hbm.at[pl.ds(base + k * CH_H, CH_H)],
                         ibufs[p], isems[p])

    def wait_idx(k, p):
        pltpu.make_async_copy(dst_hbm.at[pl.ds(base + k * CH_H, CH_H)],
                              ibufs[p], isems[p]).wait()

    pltpu.async_copy(zeros_hbm, cnt_v, sz)
    issue_idx(0, 0)
    pltpu.make_async_copy(zeros_hbm, cnt_v, sz).wait()
    ones16 = jnp.ones((16,), jnp.float32)

    def count(p):
        iv = ibufs[p]

        def body(i, carry):
            for u in range(5):
                sl = pl.ds(pl.multiple_of(i * 80 + u * 16, 16), 16)
                plsc.addupdate_scatter(cnt_v, [iv[sl]], ones16)
            return carry

        lax.fori_loop(0, CH_H // 80, body, 0)

    def super_step(i, carry):
        k0 = i * 2
        wait_idx(k0, 0)
        issue_idx(k0 + 1, 1)
        count(0)
        wait_idx(k0 + 1, 1)

        @pl.when(i < N_CHUNKS_H // 2 - 1)
        def _():
            issue_idx(k0 + 2, 0)

        count(1)
        return carry

    lax.fori_loop(0, N_CHUNKS_H // 2, super_step, 0)
    pltpu.sync_copy(cnt_v, out_hbm.at[pl.ds(w * N_PAD, N_PAD)])


@functools.partial(
    pl.kernel,
    out_type=(
        jax.ShapeDtypeStruct((NC * N_PAD,), jnp.float32),
        jax.ShapeDtypeStruct((NC * N_PAD,), jnp.float32),
    ),
    mesh=_mesh,
    scratch_types=[
        pltpu.VMEM((N_PAD,), jnp.float32),
        pltpu.VMEM((CH,), jnp.int32),
        pltpu.VMEM((CH,), jnp.int32),
        pltpu.VMEM((CH,), jnp.int32),
        pltpu.VMEM((CH,), jnp.int32),
        pltpu.VMEM((CH,), jnp.float32),
        pltpu.VMEM((CH,), jnp.float32),
        pltpu.VMEM_SHARED((N_PAD,), jnp.float32),
        pltpu.VMEM_SHARED((N_PAD,), jnp.float32),
        pltpu.SemaphoreType.DMA,
        pltpu.SemaphoreType.DMA,
        pltpu.SemaphoreType.DMA,
        pltpu.SemaphoreType.DMA,
    ],
    compiler_params=pltpu.CompilerParams(needs_layout_passes=False),
)
def _edgepass(src_hbm, dst_hbm, y_hbm, dis_hbm, zeros_hbm,
              a_out, c_out, table_v, gidx0, gidx1, sidx0, sidx1,
              vals0, vals1, a_sh, c_sh, six0, six1, ssc0, ssc1):
    c = lax.axis_index("c")
    s = lax.axis_index("s")

    @pl.when(s == 0)
    def _():
        pltpu.sync_copy(zeros_hbm, a_sh)
        pltpu.sync_copy(zeros_hbm, c_sh)

    plsc.subcore_barrier()
    base = (c * NS + s) * EPW

    def sweep(table_hbm, gather_idx_hbm, scatter_idx_hbm, acc_sh):
        pltpu.sync_copy(table_hbm, table_v)
        gbufs = (gidx0, gidx1)
        sbufs = (sidx0, sidx1)
        vbufs = (vals0, vals1)
        isems = (six0, six1)
        ssems = (ssc0, ssc1)

        def issue_idx(k, p):
            off = base + k * CH
            pltpu.async_copy(gather_idx_hbm.at[pl.ds(off, CH)],
                             gbufs[p], isems[p])
            pltpu.async_copy(scatter_idx_hbm.at[pl.ds(off, CH)],
                             sbufs[p], isems[p])

        def wait_idx(k, p):
            off = base + k * CH
            pltpu.make_async_copy(gather_idx_hbm.at[pl.ds(off, CH)],
                                  gbufs[p], isems[p]).wait()
            pltpu.make_async_copy(scatter_idx_hbm.at[pl.ds(off, CH)],
                                  sbufs[p], isems[p]).wait()

        def gather(p):
            gv, vv = gbufs[p], vbufs[p]

            def gather80(i, carry):
                for u in range(5):
                    sl = pl.ds(pl.multiple_of(i * 80 + u * 16, 16), 16)
                    vv[sl] = plsc.load_gather(table_v, [gv[sl]])
                return carry

            lax.fori_loop(0, CH // 80, gather80, 0)

        def issue_scatter(p):
            pltpu.async_copy(vbufs[p], acc_sh.at[sbufs[p]], ssems[p], add=True)

        def wait_scatter(p):
            pltpu.make_async_copy(vbufs[p], acc_sh.at[sbufs[p]],
                                  ssems[p]).wait()

        issue_idx(0, 0)

        def super_step(i, carry):
            k0 = i * 2
            wait_idx(k0, 0)
            gather(0)

            @pl.when(i > 0)
            def _():
                wait_scatter(1)

            issue_scatter(0)
            issue_idx(k0 + 1, 1)

            wait_idx(k0 + 1, 1)
            gather(1)
            wait_scatter(0)
            issue_scatter(1)

            @pl.when(i < N_CHUNKS // 2 - 1)
            def _():
                issue_idx(k0 + 2, 0)

            return carry

        lax.fori_loop(0, N_CHUNKS // 2, super_step, 0)
        wait_scatter(1)

    sweep(y_hbm, src_hbm, dst_hbm, a_sh)
    sweep(dis_hbm, dst_hbm, src_hbm, c_sh)

    plsc.subcore_barrier()
    sl = pl.ds(s * TILE_N, TILE_N)
    off = c * N_PAD + s * TILE_N
    pltpu.sync_copy(a_sh.at[sl], a_out.at[pl.ds(off, TILE_N)])
    pltpu.sync_copy(c_sh.at[sl], c_out.at[pl.ds(off, TILE_N)])


def _pw_body(cntw_ref, x_ref, dis_ref, y_ref):
    i = pl.program_id(0)

    @pl.when(i == 0)
    def _():
        dis_ref[...] = cntw_ref[...]

    @pl.when(i > 0)
    def _():
        dis_ref[...] += cntw_ref[...]

    @pl.when(i == NW - 1)
    def _():
        dis = lax.rsqrt(dis_ref[...] + 1.0)
        dis_ref[...] = dis
        y_ref[...] = x_ref[...] * dis


_pw = pl.pallas_call(
    _pw_body,
    grid=(NW,),
    in_specs=[
        pl.BlockSpec((N_PAD,), lambda i: (i,)),
        pl.BlockSpec((N_PAD,), lambda i: (0,)),
    ],
    out_specs=[
        pl.BlockSpec((N_PAD,), lambda i: (0,)),
        pl.BlockSpec((N_PAD,), lambda i: (0,)),
    ],
    out_shape=(
        jax.ShapeDtypeStruct((N_PAD,), jnp.float32),
        jax.ShapeDtypeStruct((N_PAD,), jnp.float32),
    ),
)


def _final_body(x_ref, dis_ref, a2_ref, c2_ref, mask_ref, w1_ref, b1_ref,
                w2_ref, b2_ref, wo_ref, bo_ref, out_ref):
    a = a2_ref[pl.ds(0, N_PAD)] + a2_ref[pl.ds(N_PAD, N_PAD)]
    cc = c2_ref[pl.ds(0, N_PAD)] + c2_ref[pl.ds(N_PAD, N_PAD)]
    dis = dis_ref[...]
    d2 = dis * dis
    s1 = dis * a + d2 * x_ref[...]
    t = (dis * cc + d2) * mask_ref[...]
    pooled = b2_ref[...]
    inv_n = 1.0 / N_NODES
    for j in range(16):
        h = jnp.maximum(s1 * w1_ref[0, j] + b1_ref[0, j], 0.0)
        pooled = pooled + (jnp.sum(t * h) * inv_n) * w2_ref[pl.ds(j, 1), :]
    out_ref[...] = jnp.sum(pooled * wo_ref[...]).reshape(1, 1) + bo_ref[...]


_final = pl.pallas_call(
    _final_body,
    out_shape=jax.ShapeDtypeStruct((1, 1), jnp.float32),
)


def kernel(x, edge_index, W1, b1, W2, b2, W_out, b_out):
    src = edge_index[0].astype(jnp.int32)
    dst = edge_index[1].astype(jnp.int32)
    zeros = jnp.zeros((N_PAD,), jnp.float32)
    cntw = _hist(dst, zeros)
    x_pad = jnp.pad(x[:, 0], (0, N_PAD - N_NODES))
    dis, y = _pw(cntw, x_pad)
    a2, c2 = _edgepass(src, dst, y, dis, zeros)
    mask = (jnp.arange(N_PAD) < N_NODES).astype(jnp.float32)
    return _final(
        x_pad, dis, a2, c2, mask,
        W1, b1.reshape(1, 16), W2, b2.reshape(1, 32),
        W_out.reshape(1, 32), b_out.reshape(1, 1),
    )


# speedup vs baseline: 1.0275x; 1.0275x over previous
import functools

import jax
import jax.numpy as jnp
from jax import lax
from jax.experimental import pallas as pl
from jax.experimental.pallas import tpu as pltpu
from jax.experimental.pallas import tpu_sc as plsc

N_NODES = 100000
N_EDGES = 6400000
NC, NS = 2, 16
NW = NC * NS
N_PAD = 102400
EPW = N_EDGES // NW
CH = 2000
N_CHUNKS = EPW // CH
CH_H = 4000
N_CHUNKS_H = EPW // CH_H
TILE_N = N_PAD // NS

_mesh = plsc.VectorSubcoreMesh(
    core_axis_name="c", subcore_axis_name="s", num_cores=NC, num_subcores=NS
)


@functools.partial(
    pl.kernel,
    out_type=jax.ShapeDtypeStruct((NW * N_PAD,), jnp.float32),
    mesh=_mesh,
    scratch_types=[
        pltpu.VMEM((N_PAD,), jnp.float32),
        pltpu.VMEM((CH_H,), jnp.int32),
        pltpu.VMEM((CH_H,), jnp.int32),
        pltpu.SemaphoreType.DMA,
        pltpu.SemaphoreType.DMA,
        pltpu.SemaphoreType.DMA,
    ],
    compiler_params=pltpu.CompilerParams(needs_layout_passes=False),
)
def _hist(dst_hbm, zeros_hbm, out_hbm, cnt_v, idx0, idx1, sz, si0, si1):
    c = lax.axis_index("c")
    s = lax.axis_index("s")
    w = c * NS + s
    base = w * EPW
    ibufs = (idx0, idx1)
    isems = (si0, si1)

    def issue_idx(k, p):
        pltpu.async_copy(dst_hbm.at[pl.ds(base + k * CH_H, CH_H)],
                         ibufs[p], isems[p])

    def wait_idx(k, p):
        pltpu.make_async_copy(dst_hbm.at[pl.ds(base + k * CH_H, CH_H)],
                              ibufs[p], isems[p]).wait()

    pltpu.async_copy(zeros_hbm, cnt_v, sz)
    issue_idx(0, 0)
    pltpu.make_async_copy(zeros_hbm, cnt_v, sz).wait()
    ones16 = jnp.ones((16,), jnp.float32)

    def count(p):
        iv = ibufs[p]

        def body(i, carry):
            for u in range(5):
                sl = pl.ds(pl.multiple_of(i * 80 + u * 16, 16), 16)
                plsc.addupdate_scatter(cnt_v, [iv[sl]], ones16)
            return carry

        lax.fori_loop(0, CH_H // 80, body, 0)

    def super_step(i, carry):
        k0 = i * 2
        wait_idx(k0, 0)
        issue_idx(k0 + 1, 1)
        count(0)
        wait_idx(k0 + 1, 1)

        @pl.when(i < N_CHUNKS_H // 2 - 1)
        def _():
            issue_idx(k0 + 2, 0)

        count(1)
        return carry

    lax.fori_loop(0, N_CHUNKS_H // 2, super_step, 0)
    pltpu.sync_copy(cnt_v, out_hbm.at[pl.ds(w * N_PAD, N_PAD)])


@functools.partial(
    pl.kernel,
    out_type=(
        jax.ShapeDtypeStruct((NC * N_PAD,), jnp.float32),
        jax.ShapeDtypeStruct((NC * N_PAD,), jnp.float32),
    ),
    mesh=_mesh,
    scratch_types=[
        pltpu.VMEM((N_PAD,), jnp.float32),
        pltpu.VMEM((CH,), jnp.int32),
        pltpu.VMEM((CH,), jnp.int32),
        pltpu.VMEM((CH,), jnp.int32),
        pltpu.VMEM((CH,), jnp.int32),
        pltpu.VMEM((CH,), jnp.float32),
        pltpu.VMEM((CH,), jnp.float32),
        pltpu.VMEM_SHARED((N_PAD,), jnp.float32),
        pltpu.VMEM_SHARED((N_PAD,), jnp.float32),
        pltpu.SemaphoreType.DMA,
        pltpu.SemaphoreType.DMA,
        pltpu.SemaphoreType.DMA,
        pltpu.SemaphoreType.DMA,
    ],
    compiler_params=pltpu.CompilerParams(needs_layout_passes=False),
)
def _edgepass(src_hbm, dst_hbm, y_hbm, dis_hbm, zeros_hbm,
              a_out, c_out, table_v, gidx0, gidx1, sidx0, sidx1,
              vals0, vals1, a_sh, c_sh, six0, six1, ssc0, ssc1):
    c = lax.axis_index("c")
    s = lax.axis_index("s")

    @pl.when(s == 0)
    def _():
        pltpu.sync_copy(zeros_hbm, a_sh)
        pltpu.sync_copy(zeros_hbm, c_sh)

    plsc.subcore_barrier()
    base = (c * NS + s) * EPW

    def sweep(table_hbm, gather_idx_hbm, scatter_idx_hbm, acc_sh):
        pltpu.sync_copy(table_hbm, table_v)
        gbufs = (gidx0, gidx1)
        sbufs = (sidx0, sidx1)
        vbufs = (vals0, vals1)
        isems = (six0, six1)
        ssems = (ssc0, ssc1)

        def issue_idx(k, p):
            off = base + k * CH
            pltpu.async_copy(gather_idx_hbm.at[pl.ds(off, CH)],
                             gbufs[p], isems[p])
            pltpu.async_copy(scatter_idx_hbm.at[pl.ds(off, CH)],
                             sbufs[p], isems[p])

        def wait_idx(k, p):
            off = base + k * CH
            pltpu.make_async_copy(gather_idx_hbm.at[pl.ds(off, CH)],
                                  gbufs[p], isems[p]).wait()
            pltpu.make_async_copy(scatter_idx_hbm.at[pl.ds(off, CH)],
                                  sbufs[p], isems[p]).wait()

        def gather(p):
            gv, vv = gbufs[p], vbufs[p]

            def gather80(i, carry):
                for u in range(5):
                    sl = pl.ds(pl.multiple_of(i * 80 + u * 16, 16), 16)
                    vv[sl] = plsc.load_gather(table_v, [gv[sl]])
                return carry

            lax.fori_loop(0, CH // 80, gather80, 0)

        def issue_scatter(p):
            pltpu.async_copy(vbufs[p], acc_sh.at[sbufs[p]], ssems[p], add=True)

        def wait_scatter(p):
            pltpu.make_async_copy(vbufs[p], acc_sh.at[sbufs[p]],
                                  ssems[p]).wait()

        issue_idx(0, 0)

        def super_step(i, carry):
            k0 = i * 2
            wait_idx(k0, 0)
            gather(0)

            @pl.when(i > 0)
            def _():
                wait_scatter(1)

            issue_scatter(0)
            issue_idx(k0 + 1, 1)

            wait_idx(k0 + 1, 1)
            gather(1)
            wait_scatter(0)
            issue_scatter(1)

            @pl.when(i < N_CHUNKS // 2 - 1)
            def _():
                issue_idx(k0 + 2, 0)

            return carry

        lax.fori_loop(0, N_CHUNKS // 2, super_step, 0)
        wait_scatter(1)

    sweep(y_hbm, src_hbm, dst_hbm, a_sh)
    sweep(dis_hbm, dst_hbm, src_hbm, c_sh)

    plsc.subcore_barrier()
    sl = pl.ds(s * TILE_N, TILE_N)
    off = c * N_PAD + s * TILE_N
    pltpu.sync_copy(a_sh.at[sl], a_out.at[pl.ds(off, TILE_N)])
    pltpu.sync_copy(c_sh.at[sl], c_out.at[pl.ds(off, TILE_N)])


PW_STEPS = 4
PW_PER = NW // PW_STEPS


def _pw_body(cntw_ref, x_ref, dis_ref, y_ref):
    i = pl.program_id(0)
    part = cntw_ref[pl.ds(0, N_PAD)]
    for k in range(1, PW_PER):
        part = part + cntw_ref[pl.ds(k * N_PAD, N_PAD)]

    @pl.when(i == 0)
    def _():
        dis_ref[...] = part

    @pl.when(i > 0)
    def _():
        dis_ref[...] += part

    @pl.when(i == PW_STEPS - 1)
    def _():
        dis = lax.rsqrt(dis_ref[...] + 1.0)
        dis_ref[...] = dis
        y_ref[...] = x_ref[...] * dis


_pw = pl.pallas_call(
    _pw_body,
    grid=(PW_STEPS,),
    in_specs=[
        pl.BlockSpec((PW_PER * N_PAD,), lambda i: (i,)),
        pl.BlockSpec((N_PAD,), lambda i: (0,)),
    ],
    out_specs=[
        pl.BlockSpec((N_PAD,), lambda i: (0,)),
        pl.BlockSpec((N_PAD,), lambda i: (0,)),
    ],
    out_shape=(
        jax.ShapeDtypeStruct((N_PAD,), jnp.float32),
        jax.ShapeDtypeStruct((N_PAD,), jnp.float32),
    ),
)


def _final_body(x_ref, dis_ref, a2_ref, c2_ref, mask_ref, w1_ref, b1_ref,
                w2_ref, b2_ref, wo_ref, bo_ref, out_ref):
    a = a2_ref[pl.ds(0, N_PAD)] + a2_ref[pl.ds(N_PAD, N_PAD)]
    cc = c2_ref[pl.ds(0, N_PAD)] + c2_ref[pl.ds(N_PAD, N_PAD)]
    dis = dis_ref[...]
    d2 = dis * dis
    s1 = dis * a + d2 * x_ref[...]
    t = (dis * cc + d2) * mask_ref[...]
    pooled = b2_ref[...]
    inv_n = 1.0 / N_NODES
    for j in range(16):
        h = jnp.maximum(s1 * w1_ref[0, j] + b1_ref[0, j], 0.0)
        pooled = pooled + (jnp.sum(t * h) * inv_n) * w2_ref[pl.ds(j, 1), :]
    out_ref[...] = jnp.sum(pooled * wo_ref[...]).reshape(1, 1) + bo_ref[...]


_final = pl.pallas_call(
    _final_body,
    out_shape=jax.ShapeDtypeStruct((1, 1), jnp.float32),
)


def kernel(x, edge_index, W1, b1, W2, b2, W_out, b_out):
    src = edge_index[0].astype(jnp.int32)
    dst = edge_index[1].astype(jnp.int32)
    zeros = jnp.zeros((N_PAD,), jnp.float32)
    cntw = _hist(dst, zeros)
    x_pad = jnp.pad(x[:, 0], (0, N_PAD - N_NODES))
    dis, y = _pw(cntw, x_pad)
    a2, c2 = _edgepass(src, dst, y, dis, zeros)
    mask = (jnp.arange(N_PAD) < N_NODES).astype(jnp.float32)
    return _final(
        x_pad, dis, a2, c2, mask,
        W1, b1.reshape(1, 16), W2, b2.reshape(1, 32),
        W_out.reshape(1, 32), b_out.reshape(1, 1),
    )
